# Initial kernel scaffold; baseline (speedup 1.0000x reference)
import jax
import jax.numpy as jnp
from jax.experimental import pallas as pl


def kernel(x, edge_index, batch, W1, b1, W2, b2, W3, b3):
    raise NotImplementedError("write your pallas kernel here")


# speedup vs baseline: 17.8889x; 17.8889x over previous
import functools

import jax
import jax.numpy as jnp
from jax import lax
from jax.experimental import pallas as pl
from jax.experimental.pallas import tpu as pltpu
from jax.experimental.pallas import tpu_sc as plsc

N = 10000
E = 640000
NG = 64
HID = 128
OUT = 512

NP = 10240
NC, NS = 2, 16
NW = NC * NS
K = 128
NCHUNK = 157
EPT = NCHUNK * K
EP = NW * EPT
RPT = NP // NS
RB = 2048


def _mesh():
    return plsc.VectorSubcoreMesh(core_axis_name="c", subcore_axis_name="s")


def _build_deg():
    @functools.partial(
        pl.kernel,
        out_type=jax.ShapeDtypeStruct((NC * NP, 16), jnp.float32),
        mesh=_mesh(),
        compiler_params=pltpu.CompilerParams(use_tc_tiling_on_sc=False),
        scratch_types=[
            pltpu.VMEM((K,), jnp.int32),
            pltpu.VMEM((K, 16), jnp.float32),
            pltpu.VMEM_SHARED((NP, 16), jnp.float32),
        ],
    )
    def deg_kernel(dst_hbm, ones_hbm, z_hbm, out_hbm, didx, rows, acc):
        cid = lax.axis_index("c")
        sid = lax.axis_index("s")
        r0 = sid * RPT
        pltpu.sync_copy(z_hbm.at[pl.ds(r0, RPT)], acc.at[pl.ds(r0, RPT)])
        pltpu.sync_copy(ones_hbm, rows)
        plsc.subcore_barrier()
        base = (cid * NS + sid) * EPT

        def chunk(i, c):
            off = pl.multiple_of(base + i * K, K)
            pltpu.sync_copy(dst_hbm.at[pl.ds(off, K)], didx)
            pltpu.sync_copy(rows, acc.at[didx], add=True)
            return c

        lax.fori_loop(0, NCHUNK, chunk, 0)
        plsc.subcore_barrier()
        pltpu.sync_copy(acc.at[pl.ds(r0, RPT)],
                        out_hbm.at[pl.ds(cid * NP + r0, RPT)])

    return deg_kernel


def _build_agg(F):
    @functools.partial(
        pl.kernel,
        out_type=jax.ShapeDtypeStruct((NC * NP, F), jnp.float32),
        mesh=_mesh(),
        compiler_params=pltpu.CompilerParams(use_tc_tiling_on_sc=False),
        scratch_types=[
            pltpu.VMEM((K,), jnp.int32),
            pltpu.VMEM((K,), jnp.int32),
            pltpu.VMEM((K, F), jnp.float32),
            pltpu.VMEM_SHARED((NP, F), jnp.float32),
            pltpu.SemaphoreType.DMA,
        ],
    )
    def agg_kernel(src_hbm, dst_hbm, p_hbm, z_hbm, out_hbm,
                   sidx, didx, rows, acc, sem):
        cid = lax.axis_index("c")
        sid = lax.axis_index("s")
        r0 = sid * RPT
        pltpu.sync_copy(z_hbm.at[pl.ds(r0, RPT)], acc.at[pl.ds(r0, RPT)])
        plsc.subcore_barrier()
        base = (cid * NS + sid) * EPT

        def chunk(i, c):
            off = pl.multiple_of(base + i * K, K)
            pltpu.sync_copy(src_hbm.at[pl.ds(off, K)], sidx)
            pltpu.sync_copy(dst_hbm.at[pl.ds(off, K)], didx)
            pltpu.async_copy(p_hbm.at[sidx], rows, sem).wait()
            pltpu.sync_copy(rows, acc.at[didx], add=True)
            return c

        lax.fori_loop(0, NCHUNK, chunk, 0)
        plsc.subcore_barrier()
        pltpu.sync_copy(acc.at[pl.ds(r0, RPT)],
                        out_hbm.at[pl.ds(cid * NP + r0, RPT)])

    return agg_kernel


def _build_norm():
    def body(da, db, xr, dinv_o, p1_o):
        deg = da[...] + db[...] + 1.0
        dinv = lax.rsqrt(deg)
        dinv_o[...] = dinv
        p1_o[...] = xr[...] * dinv

    return pl.pallas_call(
        body,
        out_shape=(jax.ShapeDtypeStruct((NP, 16), jnp.float32),
                   jax.ShapeDtypeStruct((NP, 16), jnp.float32)),
    )


def _build_mm(F_in, F_out):
    def body(aa, ab, p, dv, W, b, o):
        d = dv[...][:, :1]
        m = (aa[...] + ab[...] + p[...]) * d
        h = jnp.dot(m, W[...], preferred_element_type=jnp.float32) + b[...]
        o[...] = jnp.maximum(h, 0.0) * d

    rb = pl.BlockSpec((RB, F_in), lambda i: (i, 0))
    return pl.pallas_call(
        body,
        grid=(NP // RB,),
        in_specs=[rb, rb, rb,
                  pl.BlockSpec((RB, 16), lambda i: (i, 0)),
                  pl.BlockSpec((F_in, F_out), lambda i: (0, 0)),
                  pl.BlockSpec((1, F_out), lambda i: (0, 0))],
        out_specs=pl.BlockSpec((RB, F_out), lambda i: (i, 0)),
        out_shape=jax.ShapeDtypeStruct((NP, F_out), jnp.float32),
    )


def _build_final():
    G = NP // RB

    def body(aa, ab, p, dv, bt, W, b, o, seg, cnt):
        i = pl.program_id(0)

        @pl.when(i == 0)
        def _():
            seg[...] = jnp.zeros_like(seg)
            cnt[...] = jnp.zeros_like(cnt)

        g = (aa[...] + ab[...] + p[...]) * dv[...][:, :1]
        rows = i * RB + lax.broadcasted_iota(jnp.int32, (RB, 1), 0)
        valid = (rows < N).astype(jnp.float32)
        oh = (bt[...] == lax.broadcasted_iota(jnp.int32, (1, NG), 1))
        oh = oh.astype(jnp.float32) * valid
        seg[...] += lax.dot_general(oh, g, (((0,), (0,)), ((), ())),
                                    preferred_element_type=jnp.float32)
        cnt[...] += lax.dot_general(oh, valid, (((0,), (0,)), ((), ())),
                                    preferred_element_type=jnp.float32)

        @pl.when(i == G - 1)
        def _():
            pooled = seg[...] / jnp.maximum(cnt[...], 1.0)
            o[...] = jnp.dot(pooled, W[...],
                             preferred_element_type=jnp.float32) + b[...]

    rb128 = pl.BlockSpec((RB, HID), lambda i: (i, 0))
    return pl.pallas_call(
        body,
        grid=(G,),
        in_specs=[rb128, rb128, rb128,
                  pl.BlockSpec((RB, 16), lambda i: (i, 0)),
                  pl.BlockSpec((RB, 1), lambda i: (i, 0)),
                  pl.BlockSpec((HID, OUT), lambda i: (0, 0)),
                  pl.BlockSpec((1, OUT), lambda i: (0, 0))],
        out_specs=pl.BlockSpec((NG, OUT), lambda i: (0, 0)),
        out_shape=jax.ShapeDtypeStruct((NG, OUT), jnp.float32),
        scratch_shapes=[pltpu.VMEM((NG, HID), jnp.float32),
                        pltpu.VMEM((NG, 1), jnp.float32)],
    )


_deg_k = _build_deg()
_agg16 = _build_agg(16)
_agg128 = _build_agg(HID)
_norm_k = _build_norm()
_mm1 = _build_mm(16, HID)
_mm2 = _build_mm(HID, HID)
_final_k = _build_final()


def kernel(x, edge_index, batch, W1, b1, W2, b2, W3, b3):
    f32 = jnp.float32
    pad_e = jnp.full((EP - E,), N, jnp.int32)
    srcp = jnp.concatenate([edge_index[0], pad_e])
    dstp = jnp.concatenate([edge_index[1], pad_e])
    x16 = jnp.zeros((NP, 16), f32).at[:N, :8].set(x)
    batchp = jnp.pad(batch, (0, NP - N)).reshape(NP, 1)
    z16 = jnp.zeros((NP, 16), f32)
    z128 = jnp.zeros((NP, HID), f32)
    ones16 = jnp.ones((K, 16), f32)
    W1p = jnp.zeros((16, HID), f32).at[:8].set(W1)

    deg = _deg_k(dstp, ones16, z16).reshape(NC, NP, 16)
    dinv, p1 = _norm_k(deg[0], deg[1], x16)
    a1 = _agg16(srcp, dstp, p1, z16).reshape(NC, NP, 16)
    p2 = _mm1(a1[0], a1[1], p1, dinv, W1p, b1.reshape(1, HID))
    a2 = _agg128(srcp, dstp, p2, z128).reshape(NC, NP, HID)
    p3 = _mm2(a2[0], a2[1], p2, dinv, W2, b2.reshape(1, HID))
    a3 = _agg128(srcp, dstp, p3, z128).reshape(NC, NP, HID)
    return _final_k(a3[0], a3[1], p3, dinv, batchp, W3, b3.reshape(1, OUT))
